# Initial kernel scaffold; baseline (speedup 1.0000x reference)
import jax
import jax.numpy as jnp
from jax.experimental import pallas as pl


def kernel(features, edge_index, W1, b1, W2, b2):
    raise NotImplementedError("write your pallas kernel here")


# speedup vs baseline: 5.6018x; 5.6018x over previous
import functools

import jax
import jax.numpy as jnp
from jax import lax
from jax.experimental import pallas as pl
from jax.experimental.pallas import tpu as pltpu
from jax.experimental.pallas import tpu_sc as plsc

NN = 10000
NE = 320000
DF = 128
NH = 64
NCLS = 16

NC = 2
NS = 16
NW = NC * NS
EPW = NE // NW
CHUNK = 80
NITER = EPW // CHUNK
NPAD = 10240
RPT = NPAD // NS

_SC_MESH = plsc.VectorSubcoreMesh(core_axis_name="c", subcore_axis_name="s")
_SC_PARAMS = pltpu.CompilerParams(use_tc_tiling_on_sc=False)


def _deg_kernel(src_hbm, dst_hbm, dego_hbm, degi_hbm, idx_v, ones_v, zbuf,
                dego_sh, degi_sh):
    c = lax.axis_index("c")
    s = lax.axis_index("s")
    wid = c * NS + s

    def fill(i, carry):
        ones_v[pl.ds(i * 16, 16)] = jnp.ones((16,), jnp.float32)
        return carry

    lax.fori_loop(0, CHUNK // 16, fill, 0)

    def zfill(i, carry):
        zbuf[pl.ds(i * 16, 16)] = jnp.zeros((16,), jnp.float32)
        return carry

    lax.fori_loop(0, RPT // 16, zfill, 0)
    pltpu.sync_copy(zbuf, dego_sh.at[pl.ds(s * RPT, RPT)])
    pltpu.sync_copy(zbuf, degi_sh.at[pl.ds(s * RPT, RPT)])
    plsc.subcore_barrier()

    def body(j, carry):
        off = wid * EPW + j * CHUNK
        pltpu.sync_copy(src_hbm.at[pl.ds(off, CHUNK)], idx_v)
        pltpu.sync_copy(ones_v, dego_sh.at[idx_v], add=True)
        pltpu.sync_copy(dst_hbm.at[pl.ds(off, CHUNK)], idx_v)
        pltpu.sync_copy(ones_v, degi_sh.at[idx_v], add=True)
        return carry

    lax.fori_loop(0, NITER, body, 0)
    plsc.subcore_barrier()

    pltpu.sync_copy(dego_sh.at[pl.ds(s * RPT, RPT)],
                    dego_hbm.at[c, 0, pl.ds(s * RPT, RPT)])
    pltpu.sync_copy(degi_sh.at[pl.ds(s * RPT, RPT)],
                    degi_hbm.at[c, 0, pl.ds(s * RPT, RPT)])


_deg_call = pl.kernel(
    _deg_kernel,
    out_type=[jax.ShapeDtypeStruct((NC, 1, NPAD), jnp.float32),
              jax.ShapeDtypeStruct((NC, 1, NPAD), jnp.float32)],
    mesh=_SC_MESH,
    compiler_params=_SC_PARAMS,
    scratch_types=[
        pltpu.VMEM((CHUNK,), jnp.int32),
        pltpu.VMEM((CHUNK,), jnp.float32),
        pltpu.VMEM((RPT,), jnp.float32),
        pltpu.VMEM_SHARED((NPAD,), jnp.float32),
        pltpu.VMEM_SHARED((NPAD,), jnp.float32),
    ],
)


def _edge_kernel(F, src_hbm, dst_hbm, tab_hbm, acc_hbm,
                 idxs_v, idxd_v, rows_v, zbuf, acc_sh, sem):
    c = lax.axis_index("c")
    s = lax.axis_index("s")
    wid = c * NS + s

    zrows = 128
    z16 = jnp.zeros((16,), jnp.float32)

    def zero_row(i, carry):
        for jj in range(F // 16):
            zbuf[i, pl.ds(jj * 16, 16)] = z16
        return carry

    lax.fori_loop(0, zrows, zero_row, 0)

    def zero_slice(k, carry):
        pltpu.sync_copy(zbuf, acc_sh.at[pl.ds(s * RPT + k * zrows, zrows)])
        return carry

    lax.fori_loop(0, RPT // zrows, zero_slice, 0)
    plsc.subcore_barrier()

    def body(j, carry):
        off = wid * EPW + j * CHUNK
        pltpu.sync_copy(src_hbm.at[pl.ds(off, CHUNK)], idxs_v)
        pltpu.sync_copy(dst_hbm.at[pl.ds(off, CHUNK)], idxd_v)
        pltpu.async_copy(tab_hbm.at[idxs_v], rows_v, sem).wait()
        pltpu.sync_copy(rows_v, acc_sh.at[idxd_v], add=True)
        return carry

    lax.fori_loop(0, NITER, body, 0)
    plsc.subcore_barrier()

    pltpu.sync_copy(acc_sh.at[pl.ds(s * RPT, RPT)],
                    acc_hbm.at[c, pl.ds(s * RPT, RPT)])


def _make_edge_call(F):
    return pl.kernel(
        functools.partial(_edge_kernel, F),
        out_type=jax.ShapeDtypeStruct((NC, NPAD, F), jnp.float32),
        mesh=_SC_MESH,
        compiler_params=_SC_PARAMS,
        scratch_types=[
            pltpu.VMEM((CHUNK,), jnp.int32),
            pltpu.VMEM((CHUNK,), jnp.int32),
            pltpu.VMEM((CHUNK, F), jnp.float32),
            pltpu.VMEM((128, F), jnp.float32),
            pltpu.VMEM_SHARED((NPAD, F), jnp.float32),
            pltpu.SemaphoreType.DMA,
        ],
    )


_edge_call_h = _make_edge_call(NH)
_edge_call_c = _make_edge_call(NCLS)


def _mm_scale_body(x_ref, w_ref, norm_ref, o_ref):
    z = jnp.dot(x_ref[...], w_ref[...], preferred_element_type=jnp.float32)
    o_ref[...] = z * norm_ref[...]


_mm_scale = pl.pallas_call(
    _mm_scale_body,
    out_shape=jax.ShapeDtypeStruct((NN, NH), jnp.float32),
)


def _mid_body(acc_ref, ni_ref, no_ref, b1_ref, w2_ref, o_ref):
    h = acc_ref[0] + acc_ref[1]
    h = jnp.maximum(h * ni_ref[...] + b1_ref[...], 0.0)
    o_ref[...] = jnp.dot(h * no_ref[...], w2_ref[...],
                         preferred_element_type=jnp.float32)


_mid = pl.pallas_call(
    _mid_body,
    out_shape=jax.ShapeDtypeStruct((NPAD, NCLS), jnp.float32),
)


def _fin_body(acc_ref, ni_ref, b2_ref, o_ref):
    o_ref[...] = ((acc_ref[0, :NN, :] + acc_ref[1, :NN, :])
                  * ni_ref[...] + b2_ref[...])


_fin = pl.pallas_call(
    _fin_body,
    out_shape=jax.ShapeDtypeStruct((NN, NCLS), jnp.float32),
)


@jax.jit
def kernel(features, edge_index, W1, b1, W2, b2):
    src = edge_index[0]
    dst = edge_index[1]

    dego, degi = _deg_call(src, dst)
    deg_out = dego[0, 0] + dego[1, 0]
    deg_in = degi[0, 0] + degi[1, 0]
    norm_out = lax.rsqrt(jnp.maximum(deg_out, 1.0)).reshape(NPAD, 1)
    norm_in = lax.rsqrt(jnp.maximum(deg_in, 1.0)).reshape(NPAD, 1)

    hw1 = _mm_scale(features, W1, norm_out[:NN])
    acc1 = _edge_call_h(src, dst, hw1)
    hw2 = _mid(acc1, norm_in, norm_out, b1.reshape(1, NH), W2)
    acc2 = _edge_call_c(src, dst, hw2)
    return _fin(acc2, norm_in[:NN], b2.reshape(1, NCLS))


# speedup vs baseline: 16.6559x; 2.9733x over previous
import functools

import jax
import jax.numpy as jnp
from jax import lax
from jax.experimental import pallas as pl
from jax.experimental.pallas import tpu as pltpu
from jax.experimental.pallas import tpu_sc as plsc

NN = 10000
NE = 320000
DF = 128
NH = 64
NCLS = 16

NC = 2
NS = 16
NW = NC * NS
EPW = NE // NW
CHUNK = 100
NITER = EPW // CHUNK
NPAD = 10240
RPT = NPAD // NS
LAG = 8

_SC_MESH = plsc.VectorSubcoreMesh(core_axis_name="c", subcore_axis_name="s")
_SC_PARAMS = pltpu.CompilerParams(use_tc_tiling_on_sc=False)


def _deg_kernel(src_hbm, dst_hbm, dego_hbm, degi_hbm, idxs_all, idxd_all,
                ones_v, zbuf, dego_sh, degi_sh, isem, ssem):
    c = lax.axis_index("c")
    s = lax.axis_index("s")
    wid = c * NS + s

    cps = pltpu.async_copy(src_hbm.at[wid], idxs_all, isem)
    cpd = pltpu.async_copy(dst_hbm.at[wid], idxd_all, isem)

    def fill(i, carry):
        ones_v[pl.ds(i * 16, 16)] = jnp.ones((16,), jnp.float32)
        return carry

    lax.fori_loop(0, CHUNK // 16 + 1, fill, 0)

    def zfill(i, carry):
        zbuf[pl.ds(i * 16, 16)] = jnp.zeros((16,), jnp.float32)
        return carry

    lax.fori_loop(0, RPT // 16, zfill, 0)
    pltpu.sync_copy(zbuf, dego_sh.at[pl.ds(s * RPT, RPT)])
    pltpu.sync_copy(zbuf, degi_sh.at[pl.ds(s * RPT, RPT)])
    cps.wait()
    cpd.wait()
    plsc.subcore_barrier()

    ones_c = ones_v.at[pl.ds(0, CHUNK)]

    def body(j, carry):
        @pl.when(j >= LAG)
        def _drain():
            pltpu.make_async_copy(ones_c, dego_sh.at[idxs_all.at[0]],
                                  ssem).wait()
            pltpu.make_async_copy(ones_c, degi_sh.at[idxd_all.at[0]],
                                  ssem).wait()

        pltpu.async_copy(ones_c, dego_sh.at[idxs_all.at[j]], ssem, add=True)
        pltpu.async_copy(ones_c, degi_sh.at[idxd_all.at[j]], ssem, add=True)
        return carry

    lax.fori_loop(0, NITER, body, 0)

    def drain(j, carry):
        pltpu.make_async_copy(ones_c, dego_sh.at[idxs_all.at[0]], ssem).wait()
        pltpu.make_async_copy(ones_c, degi_sh.at[idxd_all.at[0]], ssem).wait()
        return carry

    lax.fori_loop(0, LAG, drain, 0)
    plsc.subcore_barrier()

    pltpu.sync_copy(dego_sh.at[pl.ds(s * RPT, RPT)],
                    dego_hbm.at[c, 0, pl.ds(s * RPT, RPT)])
    pltpu.sync_copy(degi_sh.at[pl.ds(s * RPT, RPT)],
                    degi_hbm.at[c, 0, pl.ds(s * RPT, RPT)])


_deg_call = pl.kernel(
    _deg_kernel,
    out_type=[jax.ShapeDtypeStruct((NC, 1, NPAD), jnp.float32),
              jax.ShapeDtypeStruct((NC, 1, NPAD), jnp.float32)],
    mesh=_SC_MESH,
    compiler_params=_SC_PARAMS,
    scratch_types=[
        pltpu.VMEM((NITER, CHUNK), jnp.int32),
        pltpu.VMEM((NITER, CHUNK), jnp.int32),
        pltpu.VMEM((112,), jnp.float32),
        pltpu.VMEM((RPT,), jnp.float32),
        pltpu.VMEM_SHARED((NPAD,), jnp.float32),
        pltpu.VMEM_SHARED((NPAD,), jnp.float32),
        pltpu.SemaphoreType.DMA,
        pltpu.SemaphoreType.DMA,
    ],
)


def _edge_kernel(F, src_hbm, dst_hbm, tab_hbm, acc_hbm,
                 idxs_all, idxd_all, rows0, rows1, zbuf, acc_sh,
                 isem, g0, g1):
    c = lax.axis_index("c")
    s = lax.axis_index("s")
    wid = c * NS + s

    cps = pltpu.async_copy(src_hbm.at[wid], idxs_all, isem)
    cpd = pltpu.async_copy(dst_hbm.at[wid], idxd_all, isem)

    zrows = 128
    z16 = jnp.zeros((16,), jnp.float32)

    def zero_row(i, carry):
        for jj in range(F // 16):
            zbuf[i, pl.ds(jj * 16, 16)] = z16
        return carry

    lax.fori_loop(0, zrows, zero_row, 0)

    def zero_slice(k, carry):
        pltpu.sync_copy(zbuf, acc_sh.at[pl.ds(s * RPT + k * zrows, zrows)])
        return carry

    lax.fori_loop(0, RPT // zrows, zero_slice, 0)
    cps.wait()
    cpd.wait()

    pltpu.async_copy(tab_hbm.at[idxs_all.at[0]], rows0, g0)
    pltpu.async_copy(tab_hbm.at[idxs_all.at[1]], rows1, g1)
    plsc.subcore_barrier()

    def body(i, carry):
        j0 = 2 * i
        j1 = j0 + 1
        pltpu.make_async_copy(tab_hbm.at[idxs_all.at[j0]], rows0, g0).wait()
        pltpu.sync_copy(rows0, acc_sh.at[idxd_all.at[j0]], add=True)

        @pl.when(j0 + 2 < NITER)
        def _g0():
            pltpu.async_copy(tab_hbm.at[idxs_all.at[j0 + 2]], rows0, g0)

        pltpu.make_async_copy(tab_hbm.at[idxs_all.at[j1]], rows1, g1).wait()
        pltpu.sync_copy(rows1, acc_sh.at[idxd_all.at[j1]], add=True)

        @pl.when(j1 + 2 < NITER)
        def _g1():
            pltpu.async_copy(tab_hbm.at[idxs_all.at[j1 + 2]], rows1, g1)

        return carry

    lax.fori_loop(0, NITER // 2, body, 0)
    plsc.subcore_barrier()

    pltpu.sync_copy(acc_sh.at[pl.ds(s * RPT, RPT)],
                    acc_hbm.at[c, pl.ds(s * RPT, RPT)])


def _make_edge_call(F):
    return pl.kernel(
        functools.partial(_edge_kernel, F),
        out_type=jax.ShapeDtypeStruct((NC, NPAD, F), jnp.float32),
        mesh=_SC_MESH,
        compiler_params=_SC_PARAMS,
        scratch_types=[
            pltpu.VMEM((NITER, CHUNK), jnp.int32),
            pltpu.VMEM((NITER, CHUNK), jnp.int32),
            pltpu.VMEM((CHUNK, F), jnp.float32),
            pltpu.VMEM((CHUNK, F), jnp.float32),
            pltpu.VMEM((128, F), jnp.float32),
            pltpu.VMEM_SHARED((NPAD, F), jnp.float32),
            pltpu.SemaphoreType.DMA,
            pltpu.SemaphoreType.DMA,
            pltpu.SemaphoreType.DMA,
        ],
    )


_edge_call_h = _make_edge_call(NH)
_edge_call_c = _make_edge_call(NCLS)


def _mm_scale_body(x_ref, w_ref, norm_ref, o_ref):
    z = jnp.dot(x_ref[...], w_ref[...], preferred_element_type=jnp.float32)
    o_ref[...] = z * norm_ref[...]


_mm_scale = pl.pallas_call(
    _mm_scale_body,
    out_shape=jax.ShapeDtypeStruct((NN, NH), jnp.float32),
)


def _mid_body(acc_ref, ni_ref, no_ref, b1_ref, w2_ref, o_ref):
    h = acc_ref[0] + acc_ref[1]
    h = jnp.maximum(h * ni_ref[...] + b1_ref[...], 0.0)
    o_ref[...] = jnp.dot(h * no_ref[...], w2_ref[...],
                         preferred_element_type=jnp.float32)


_mid = pl.pallas_call(
    _mid_body,
    out_shape=jax.ShapeDtypeStruct((NPAD, NCLS), jnp.float32),
)


def _fin_body(acc_ref, ni_ref, b2_ref, o_ref):
    o_ref[...] = ((acc_ref[0, :NN, :] + acc_ref[1, :NN, :])
                  * ni_ref[...] + b2_ref[...])


_fin = pl.pallas_call(
    _fin_body,
    out_shape=jax.ShapeDtypeStruct((NN, NCLS), jnp.float32),
)


@jax.jit
def kernel(features, edge_index, W1, b1, W2, b2):
    src = edge_index[0].reshape(NW, NITER, CHUNK)
    dst = edge_index[1].reshape(NW, NITER, CHUNK)

    dego, degi = _deg_call(src, dst)
    deg_out = dego[0, 0] + dego[1, 0]
    deg_in = degi[0, 0] + degi[1, 0]
    norm_out = lax.rsqrt(jnp.maximum(deg_out, 1.0)).reshape(NPAD, 1)
    norm_in = lax.rsqrt(jnp.maximum(deg_in, 1.0)).reshape(NPAD, 1)

    hw1 = _mm_scale(features, W1, norm_out[:NN])
    acc1 = _edge_call_h(src, dst, hw1)
    hw2 = _mid(acc1, norm_in, norm_out, b1.reshape(1, NH), W2)
    acc2 = _edge_call_c(src, dst, hw2)
    return _fin(acc2, norm_in[:NN], b2.reshape(1, NCLS))


# speedup vs baseline: 18.3139x; 1.0995x over previous
import functools

import jax
import jax.numpy as jnp
from jax import lax
from jax.experimental import pallas as pl
from jax.experimental.pallas import tpu as pltpu
from jax.experimental.pallas import tpu_sc as plsc

NN = 10000
NE = 320000
DF = 128
NH = 64
NCLS = 16

NC = 2
NS = 16
NW = NC * NS
EPW = NE // NW
CHUNK = 100
NITER = EPW // CHUNK
NPAD = 10240
RPT = NPAD // NS
LAG = 8

_SC_MESH = plsc.VectorSubcoreMesh(core_axis_name="c", subcore_axis_name="s")
_SC_PARAMS = pltpu.CompilerParams(use_tc_tiling_on_sc=False)


def _deg_kernel(src_hbm, dst_hbm, dego_hbm, degi_hbm, idxs_all, idxd_all,
                ones_v, zbuf, dego_sh, degi_sh, isem, ssem):
    c = lax.axis_index("c")
    s = lax.axis_index("s")
    wid = c * NS + s

    cps = pltpu.async_copy(src_hbm.at[wid], idxs_all, isem)
    cpd = pltpu.async_copy(dst_hbm.at[wid], idxd_all, isem)

    def fill(i, carry):
        ones_v[pl.ds(i * 16, 16)] = jnp.ones((16,), jnp.float32)
        return carry

    lax.fori_loop(0, CHUNK // 16 + 1, fill, 0)

    def zfill(i, carry):
        zbuf[pl.ds(i * 16, 16)] = jnp.zeros((16,), jnp.float32)
        return carry

    lax.fori_loop(0, RPT // 16, zfill, 0)
    pltpu.sync_copy(zbuf, dego_sh.at[pl.ds(s * RPT, RPT)])
    pltpu.sync_copy(zbuf, degi_sh.at[pl.ds(s * RPT, RPT)])
    cps.wait()
    cpd.wait()
    plsc.subcore_barrier()

    ones_c = ones_v.at[pl.ds(0, CHUNK)]

    def body(j, carry):
        @pl.when(j >= LAG)
        def _drain():
            pltpu.make_async_copy(ones_c, dego_sh.at[idxs_all.at[0]],
                                  ssem).wait()
            pltpu.make_async_copy(ones_c, degi_sh.at[idxd_all.at[0]],
                                  ssem).wait()

        pltpu.async_copy(ones_c, dego_sh.at[idxs_all.at[j]], ssem, add=True)
        pltpu.async_copy(ones_c, degi_sh.at[idxd_all.at[j]], ssem, add=True)
        return carry

    lax.fori_loop(0, NITER, body, 0)

    def drain(j, carry):
        pltpu.make_async_copy(ones_c, dego_sh.at[idxs_all.at[0]], ssem).wait()
        pltpu.make_async_copy(ones_c, degi_sh.at[idxd_all.at[0]], ssem).wait()
        return carry

    lax.fori_loop(0, LAG, drain, 0)
    plsc.subcore_barrier()

    pltpu.sync_copy(dego_sh.at[pl.ds(s * RPT, RPT)],
                    dego_hbm.at[c, 0, pl.ds(s * RPT, RPT)])
    pltpu.sync_copy(degi_sh.at[pl.ds(s * RPT, RPT)],
                    degi_hbm.at[c, 0, pl.ds(s * RPT, RPT)])


_deg_call = pl.kernel(
    _deg_kernel,
    out_type=[jax.ShapeDtypeStruct((NC, 1, NPAD), jnp.float32),
              jax.ShapeDtypeStruct((NC, 1, NPAD), jnp.float32)],
    mesh=_SC_MESH,
    compiler_params=_SC_PARAMS,
    scratch_types=[
        pltpu.VMEM((NITER, CHUNK), jnp.int32),
        pltpu.VMEM((NITER, CHUNK), jnp.int32),
        pltpu.VMEM((112,), jnp.float32),
        pltpu.VMEM((RPT,), jnp.float32),
        pltpu.VMEM_SHARED((NPAD,), jnp.float32),
        pltpu.VMEM_SHARED((NPAD,), jnp.float32),
        pltpu.SemaphoreType.DMA,
        pltpu.SemaphoreType.DMA,
    ],
)


def _edge_kernel(F, src_hbm, dst_hbm, tab_hbm, acc_hbm,
                 idxs_all, idxd_all, rows0, rows1, zbuf, acc_sh, tab_sh,
                 isem, g0, g1):
    c = lax.axis_index("c")
    s = lax.axis_index("s")
    wid = c * NS + s

    cps = pltpu.async_copy(src_hbm.at[wid], idxs_all, isem)
    cpd = pltpu.async_copy(dst_hbm.at[wid], idxd_all, isem)
    cpt = pltpu.async_copy(tab_hbm.at[pl.ds(s * RPT, RPT)],
                           tab_sh.at[pl.ds(s * RPT, RPT)], isem)

    zrows = 128
    z16 = jnp.zeros((16,), jnp.float32)

    def zero_row(i, carry):
        for jj in range(F // 16):
            zbuf[i, pl.ds(jj * 16, 16)] = z16
        return carry

    lax.fori_loop(0, zrows, zero_row, 0)

    def zero_slice(k, carry):
        pltpu.sync_copy(zbuf, acc_sh.at[pl.ds(s * RPT + k * zrows, zrows)])
        return carry

    lax.fori_loop(0, RPT // zrows, zero_slice, 0)
    cps.wait()
    cpd.wait()
    cpt.wait()
    plsc.subcore_barrier()

    pltpu.async_copy(tab_sh.at[idxs_all.at[0]], rows0, g0)
    pltpu.async_copy(tab_sh.at[idxs_all.at[1]], rows1, g1)

    def body(i, carry):
        j0 = 2 * i
        j1 = j0 + 1
        pltpu.make_async_copy(tab_sh.at[idxs_all.at[j0]], rows0, g0).wait()
        pltpu.sync_copy(rows0, acc_sh.at[idxd_all.at[j0]], add=True)

        @pl.when(j0 + 2 < NITER)
        def _g0():
            pltpu.async_copy(tab_sh.at[idxs_all.at[j0 + 2]], rows0, g0)

        pltpu.make_async_copy(tab_sh.at[idxs_all.at[j1]], rows1, g1).wait()
        pltpu.sync_copy(rows1, acc_sh.at[idxd_all.at[j1]], add=True)

        @pl.when(j1 + 2 < NITER)
        def _g1():
            pltpu.async_copy(tab_sh.at[idxs_all.at[j1 + 2]], rows1, g1)

        return carry

    lax.fori_loop(0, NITER // 2, body, 0)
    plsc.subcore_barrier()

    pltpu.sync_copy(acc_sh.at[pl.ds(s * RPT, RPT)],
                    acc_hbm.at[c, pl.ds(s * RPT, RPT)])


def _make_edge_call(F):
    return pl.kernel(
        functools.partial(_edge_kernel, F),
        out_type=jax.ShapeDtypeStruct((NC, NPAD, F), jnp.float32),
        mesh=_SC_MESH,
        compiler_params=_SC_PARAMS,
        scratch_types=[
            pltpu.VMEM((NITER, CHUNK), jnp.int32),
            pltpu.VMEM((NITER, CHUNK), jnp.int32),
            pltpu.VMEM((CHUNK, F), jnp.float32),
            pltpu.VMEM((CHUNK, F), jnp.float32),
            pltpu.VMEM((128, F), jnp.float32),
            pltpu.VMEM_SHARED((NPAD, F), jnp.float32),
            pltpu.VMEM_SHARED((NPAD, F), jnp.float32),
            pltpu.SemaphoreType.DMA,
            pltpu.SemaphoreType.DMA,
            pltpu.SemaphoreType.DMA,
        ],
    )


_edge_call_h = _make_edge_call(NH)
_edge_call_c = _make_edge_call(NCLS)


def _mm_scale_body(x_ref, w_ref, norm_ref, o_ref):
    z = jnp.dot(x_ref[...], w_ref[...], preferred_element_type=jnp.float32)
    o_ref[0:NN, :] = z * norm_ref[...]
    o_ref[NN:NPAD, :] = jnp.zeros((NPAD - NN, NH), jnp.float32)


_mm_scale = pl.pallas_call(
    _mm_scale_body,
    out_shape=jax.ShapeDtypeStruct((NPAD, NH), jnp.float32),
)


def _mid_body(acc_ref, ni_ref, no_ref, b1_ref, w2_ref, o_ref):
    h = acc_ref[0] + acc_ref[1]
    h = jnp.maximum(h * ni_ref[...] + b1_ref[...], 0.0)
    o_ref[...] = jnp.dot(h * no_ref[...], w2_ref[...],
                         preferred_element_type=jnp.float32)


_mid = pl.pallas_call(
    _mid_body,
    out_shape=jax.ShapeDtypeStruct((NPAD, NCLS), jnp.float32),
)


def _fin_body(acc_ref, ni_ref, b2_ref, o_ref):
    o_ref[...] = ((acc_ref[0, :NN, :] + acc_ref[1, :NN, :])
                  * ni_ref[...] + b2_ref[...])


_fin = pl.pallas_call(
    _fin_body,
    out_shape=jax.ShapeDtypeStruct((NN, NCLS), jnp.float32),
)


@jax.jit
def kernel(features, edge_index, W1, b1, W2, b2):
    src = edge_index[0].reshape(NW, NITER, CHUNK)
    dst = edge_index[1].reshape(NW, NITER, CHUNK)

    dego, degi = _deg_call(src, dst)
    deg_out = dego[0, 0] + dego[1, 0]
    deg_in = degi[0, 0] + degi[1, 0]
    norm_out = lax.rsqrt(jnp.maximum(deg_out, 1.0)).reshape(NPAD, 1)
    norm_in = lax.rsqrt(jnp.maximum(deg_in, 1.0)).reshape(NPAD, 1)

    hw1 = _mm_scale(features, W1, norm_out[:NN])
    acc1 = _edge_call_h(src, dst, hw1)
    hw2 = _mid(acc1, norm_in, norm_out, b1.reshape(1, NH), W2)
    acc2 = _edge_call_c(src, dst, hw2)
    return _fin(acc2, norm_in[:NN], b2.reshape(1, NCLS))
